# speedup vs baseline: 3.1214x; 3.1214x over previous
import jax
import jax.numpy as jnp
from jax.experimental import pallas as pl
from jax.experimental.pallas import tpu as pltpu
from jax.experimental.pallas import tpu_sc as plsc

_VOCAB = 100000
_HIDDEN = 128
_B, _T = 4096, 50
_N = _B * _T
_WINDOW = 128
_GRID = _N // _WINDOW


@jax.jit
def _sc_gather(W, idx2d):
    mesh = plsc.VectorSubcoreMesh(core_axis_name="core",
                                  subcore_axis_name="subcore")

    @pl.kernel(
        out_type=jax.ShapeDtypeStruct((_N, _HIDDEN), jnp.float32),
        mesh=mesh,
    )
    def k(w_hbm, i_hbm, o_hbm):
        def body(i_vmem, o_vmem):
            pltpu.sync_copy(w_hbm.at[i_vmem.at[0]], o_vmem)

        pltpu.emit_pipeline(
            body,
            grid=(_GRID,),
            in_specs=[pl.BlockSpec((1, _WINDOW), lambda i: (0, i))],
            out_specs=[pl.BlockSpec((_WINDOW, _HIDDEN), lambda i: (i, 0))],
            core_axis_name=("core", "subcore"),
            dimension_semantics=(pltpu.PARALLEL,),
        )(i_hbm, o_hbm)

    return k(W, idx2d)


def kernel(x, W):
    idx2d = x.reshape(1, _N).astype(jnp.int32)
    out = _sc_gather(W, idx2d)
    return out.reshape(_B, _T, _HIDDEN)


# speedup vs baseline: 3.1825x; 1.0196x over previous
import jax
import jax.numpy as jnp
from jax.experimental import pallas as pl
from jax.experimental.pallas import tpu as pltpu
from jax.experimental.pallas import tpu_sc as plsc

_VOCAB = 100000
_HIDDEN = 128
_B, _T = 4096, 50
_N = _B * _T
_WINDOW = 128
_K = 2
_STEP_ROWS = _K * _WINDOW
_GRID = _N // _STEP_ROWS


@jax.jit
def _sc_gather(W, idx2d):
    mesh = plsc.VectorSubcoreMesh(core_axis_name="core",
                                  subcore_axis_name="subcore")

    @pl.kernel(
        out_type=jax.ShapeDtypeStruct((_N, _HIDDEN), jnp.float32),
        mesh=mesh,
    )
    def k(w_hbm, i_hbm, o_hbm):
        def body(i_vmem, o_vmem):
            for r in range(_K):
                pltpu.sync_copy(w_hbm.at[i_vmem.at[r]],
                                o_vmem.at[pl.ds(r * _WINDOW, _WINDOW)])

        pltpu.emit_pipeline(
            body,
            grid=(_GRID,),
            in_specs=[pl.BlockSpec((_K, _WINDOW), lambda i: (i, 0))],
            out_specs=[pl.BlockSpec((_STEP_ROWS, _HIDDEN), lambda i: (i, 0))],
            core_axis_name=("core", "subcore"),
            dimension_semantics=(pltpu.PARALLEL,),
        )(i_hbm, o_hbm)

    return k(W, idx2d)


def kernel(x, W):
    idx2d = x.reshape(_GRID * _K, _WINDOW).astype(jnp.int32)
    out = _sc_gather(W, idx2d)
    return out.reshape(_B, _T, _HIDDEN)


# speedup vs baseline: 4.1731x; 1.3113x over previous
import jax
import jax.numpy as jnp
from jax.experimental import pallas as pl
from jax.experimental.pallas import tpu as pltpu
from jax.experimental.pallas import tpu_sc as plsc

_VOCAB = 100000
_HIDDEN = 128
_B, _T = 4096, 50
_N = _B * _T
_TP = 56
_K = 2
_GRID = _B // _K


@jax.jit
def _sc_gather(W, idx_flat):
    mesh = plsc.VectorSubcoreMesh(core_axis_name="core",
                                  subcore_axis_name="subcore")

    @pl.kernel(
        out_type=jax.ShapeDtypeStruct((_B, _T, _HIDDEN), jnp.float32),
        mesh=mesh,
        compiler_params=pltpu.CompilerParams(use_tc_tiling_on_sc=True),
    )
    def k(w_hbm, i_hbm, o_hbm):
        def body(i_vmem, o_vmem):
            for r in range(_K):
                pltpu.sync_copy(w_hbm.at[i_vmem.at[pl.ds(r * _TP, _T)]],
                                o_vmem.at[r])

        pltpu.emit_pipeline(
            body,
            grid=(_GRID,),
            in_specs=[pl.BlockSpec((_K * _TP,), lambda i: (i,))],
            out_specs=[pl.BlockSpec((_K, _T, _HIDDEN), lambda i: (i, 0, 0))],
            core_axis_name=("core", "subcore"),
            dimension_semantics=(pltpu.PARALLEL,),
        )(i_hbm, o_hbm)

    return k(W, idx_flat)


def kernel(x, W):
    xi = x.astype(jnp.int32)
    idx_flat = jnp.pad(xi, ((0, 0), (0, _TP - _T))).reshape(_B * _TP)
    return _sc_gather(W, idx_flat)


# speedup vs baseline: 9.0664x; 2.1726x over previous
import jax
import jax.numpy as jnp
from jax.experimental import pallas as pl
from jax.experimental.pallas import tpu as pltpu
from jax.experimental.pallas import tpu_sc as plsc

_VOCAB = 100000
_HIDDEN = 128
_B, _T = 4096, 50
_N = _B * _T
_WINDOW = 128
_K = 2
_STEP_ROWS = _K * _WINDOW
_GRID = _N // _STEP_ROWS


@jax.jit
def _sc_gather(W, idx2d):
    mesh = plsc.VectorSubcoreMesh(core_axis_name="core",
                                  subcore_axis_name="subcore")

    @pl.kernel(
        out_type=jax.ShapeDtypeStruct((_N, _HIDDEN), jnp.float32),
        mesh=mesh,
        compiler_params=pltpu.CompilerParams(use_tc_tiling_on_sc=True),
    )
    def k(w_hbm, i_hbm, o_hbm):
        def body(i_vmem, o_vmem):
            for r in range(_K):
                pltpu.sync_copy(w_hbm.at[i_vmem.at[r]],
                                o_vmem.at[pl.ds(r * _WINDOW, _WINDOW)])

        pltpu.emit_pipeline(
            body,
            grid=(_GRID,),
            in_specs=[pl.BlockSpec((_K, _WINDOW), lambda i: (i, 0))],
            out_specs=[pl.BlockSpec((_STEP_ROWS, _HIDDEN), lambda i: (i, 0))],
            core_axis_name=("core", "subcore"),
            dimension_semantics=(pltpu.PARALLEL,),
        )(i_hbm, o_hbm)

    return k(W, idx2d)


def kernel(x, W):
    idx2d = x.T.astype(jnp.int32).reshape(_N // _WINDOW, _WINDOW)
    out = _sc_gather(W, idx2d)
    return out.reshape(_T, _B, _HIDDEN).transpose(1, 0, 2)


# speedup vs baseline: 10.4446x; 1.1520x over previous
import jax
import jax.numpy as jnp
from jax.experimental import pallas as pl
from jax.experimental.pallas import tpu as pltpu
from jax.experimental.pallas import tpu_sc as plsc

_VOCAB = 100000
_HIDDEN = 128
_B, _T = 4096, 50
_N = _B * _T
_WINDOW = 128
_K = 2
_STEP_ROWS = _K * _WINDOW
_GRID = _N // _STEP_ROWS


@jax.jit
def _sc_gather(W, idx2d):
    mesh = plsc.VectorSubcoreMesh(core_axis_name="core",
                                  subcore_axis_name="subcore")

    @pl.kernel(
        out_type=jax.ShapeDtypeStruct((_N, _HIDDEN), jnp.float32),
        mesh=mesh,
        scratch_types=[pltpu.SemaphoreType.DMA((_K,))],
        compiler_params=pltpu.CompilerParams(use_tc_tiling_on_sc=True),
    )
    def k(w_hbm, i_hbm, o_hbm, gsem):
        def body(i_vmem, o_vmem):
            handles = [
                pltpu.async_copy(w_hbm.at[i_vmem.at[r]],
                                 o_vmem.at[pl.ds(r * _WINDOW, _WINDOW)],
                                 gsem.at[r])
                for r in range(_K)
            ]
            for h in handles:
                h.wait()

        pltpu.emit_pipeline(
            body,
            grid=(_GRID,),
            in_specs=[pl.BlockSpec((_K, _WINDOW), lambda i: (i, 0))],
            out_specs=[pl.BlockSpec((_STEP_ROWS, _HIDDEN), lambda i: (i, 0))],
            core_axis_name=("core", "subcore"),
            dimension_semantics=(pltpu.PARALLEL,),
        )(i_hbm, o_hbm)

    return k(W, idx2d)


def kernel(x, W):
    idx2d = x.T.astype(jnp.int32).reshape(_N // _WINDOW, _WINDOW)
    out = _sc_gather(W, idx2d)
    return out.reshape(_T, _B, _HIDDEN).transpose(1, 0, 2)


# speedup vs baseline: 10.5361x; 1.0088x over previous
import jax
import jax.numpy as jnp
from jax import lax
from jax.experimental import pallas as pl
from jax.experimental.pallas import tpu as pltpu
from jax.experimental.pallas import tpu_sc as plsc

_VOCAB = 100000
_HIDDEN = 128
_B, _T = 4096, 50
_N = _B * _T
_WINDOW = 128
_NWIN = _N // _WINDOW
_NW = 32
_WPW = _NWIN // _NW
_G = 3
_ROUNDS = _WPW // (2 * _G)


@jax.jit
def _sc_gather(W, idx1d):
    mesh = plsc.VectorSubcoreMesh(core_axis_name="core",
                                  subcore_axis_name="subcore")

    @pl.kernel(
        out_type=jax.ShapeDtypeStruct((_NWIN, _WINDOW, _HIDDEN), jnp.float32),
        mesh=mesh,
        scratch_types=(
            [pltpu.VMEM((_WPW * _WINDOW,), jnp.int32)]
            + [pltpu.VMEM((_WINDOW, _HIDDEN), jnp.float32) for _ in range(2 * _G)]
            + [pltpu.SemaphoreType.DMA((2 * _G,)),
               pltpu.SemaphoreType.DMA((2 * _G,))]
        ),
    )
    def k(w_hbm, i_hbm, o_hbm, idx_v, b0, b1, b2, b3, b4, b5, gsem, osem):
        bufs = [b0, b1, b2, b3, b4, b5]
        cid = lax.axis_index("core")
        sid = lax.axis_index("subcore")
        wid = sid * 2 + cid
        base = wid * _WPW

        def gather_start(win, s):
            pltpu.async_copy(w_hbm.at[idx_v.at[pl.ds(win * _WINDOW, _WINDOW)]],
                             bufs[s], gsem.at[s])

        def gather_wait(win, s):
            pltpu.make_async_copy(
                w_hbm.at[idx_v.at[pl.ds(win * _WINDOW, _WINDOW)]],
                bufs[s], gsem.at[s]).wait()

        def wb_start(win, s):
            pltpu.async_copy(bufs[s], o_hbm.at[base + win], osem.at[s])

        def wb_wait(win, s):
            pltpu.make_async_copy(bufs[s], o_hbm.at[base + win],
                                  osem.at[s]).wait()

        pltpu.sync_copy(i_hbm.at[pl.ds(base * _WINDOW, _WPW * _WINDOW)], idx_v)

        for s in range(_G):
            gather_start(s, s)

        @pl.loop(0, _ROUNDS)
        def _(t):
            w0 = t * 2 * _G
            for s in range(_G):
                @pl.when(t > 0)
                def _():
                    wb_wait(w0 - _G + s, _G + s)
                gather_start(w0 + _G + s, _G + s)
            for s in range(_G):
                gather_wait(w0 + s, s)
                wb_start(w0 + s, s)
            for s in range(_G):
                wb_wait(w0 + s, s)
                @pl.when(t < _ROUNDS - 1)
                def _():
                    gather_start(w0 + 2 * _G + s, s)
            for s in range(_G):
                gather_wait(w0 + _G + s, _G + s)
                wb_start(w0 + _G + s, _G + s)

        tail0 = _ROUNDS * 2 * _G
        for i in range(_WPW - tail0):
            gather_start(tail0 + i, i)
        for s in range(_G):
            wb_wait(tail0 - _G + s, _G + s)
        for i in range(_WPW - tail0):
            gather_wait(tail0 + i, i)
            wb_start(tail0 + i, i)
        for i in range(_WPW - tail0):
            wb_wait(tail0 + i, i)

    return k(W, idx1d)


def kernel(x, W):
    idx1d = x.T.astype(jnp.int32).reshape(_N)
    out = _sc_gather(W, idx1d)
    return out.reshape(_T, _B, _HIDDEN).transpose(1, 0, 2)


# speedup vs baseline: 10.5608x; 1.0023x over previous
import jax
import jax.numpy as jnp
from jax import lax
from jax.experimental import pallas as pl
from jax.experimental.pallas import tpu as pltpu
from jax.experimental.pallas import tpu_sc as plsc

_VOCAB = 100000
_HIDDEN = 128
_B, _T = 4096, 50
_N = _B * _T
_WINDOW = 128
_NWIN = _N // _WINDOW
_NW = 32
_WPW = _NWIN // _NW
_G = 3
_ROUNDS = _WPW // (2 * _G)
_TAIL = _WPW - _ROUNDS * 2 * _G


@jax.jit
def _sc_gather(W, idx1d):
    mesh = plsc.VectorSubcoreMesh(core_axis_name="core",
                                  subcore_axis_name="subcore")

    @pl.kernel(
        out_type=jax.ShapeDtypeStruct((_N, _HIDDEN), jnp.float32),
        mesh=mesh,
        scratch_types=(
            [pltpu.VMEM((_WPW * _WINDOW,), jnp.int32)]
            + [pltpu.VMEM((_G * _WINDOW, _HIDDEN), jnp.float32)
               for _ in range(2)]
            + [pltpu.SemaphoreType.DMA((2 * _G,)),
               pltpu.SemaphoreType.DMA((2,))]
        ),
    )
    def k(w_hbm, i_hbm, o_hbm, idx_v, buf_a, buf_b, gsem, osem):
        bufs = [buf_a, buf_b]
        cid = lax.axis_index("core")
        sid = lax.axis_index("subcore")
        wid = sid * 2 + cid
        base = wid * _WPW

        def gather_start(win, grp, s):
            pltpu.async_copy(
                w_hbm.at[idx_v.at[pl.ds(win * _WINDOW, _WINDOW)]],
                bufs[grp].at[pl.ds(s * _WINDOW, _WINDOW)],
                gsem.at[grp * _G + s])

        def gather_wait(win, grp, s):
            pltpu.make_async_copy(
                w_hbm.at[idx_v.at[pl.ds(win * _WINDOW, _WINDOW)]],
                bufs[grp].at[pl.ds(s * _WINDOW, _WINDOW)],
                gsem.at[grp * _G + s]).wait()

        def wb_start(win0, grp, nwin):
            pltpu.async_copy(
                bufs[grp].at[pl.ds(0, nwin * _WINDOW)],
                o_hbm.at[pl.ds((base + win0) * _WINDOW, nwin * _WINDOW)],
                osem.at[grp])

        def wb_wait(win0, grp, nwin):
            pltpu.make_async_copy(
                bufs[grp].at[pl.ds(0, nwin * _WINDOW)],
                o_hbm.at[pl.ds((base + win0) * _WINDOW, nwin * _WINDOW)],
                osem.at[grp]).wait()

        pltpu.sync_copy(i_hbm.at[pl.ds(base * _WINDOW, _WPW * _WINDOW)], idx_v)

        for s in range(_G):
            gather_start(s, 0, s)

        @pl.loop(0, _ROUNDS)
        def _(t):
            w0 = t * 2 * _G
            @pl.when(t > 0)
            def _():
                wb_wait(w0 - _G, 1, _G)
            for s in range(_G):
                gather_start(w0 + _G + s, 1, s)
            for s in range(_G):
                gather_wait(w0 + s, 0, s)
            wb_start(w0, 0, _G)
            wb_wait(w0, 0, _G)
            for s in range(_G):
                @pl.when(t < _ROUNDS - 1)
                def _():
                    gather_start(w0 + 2 * _G + s, 0, s)
            for s in range(_G):
                gather_wait(w0 + _G + s, 1, s)
            wb_start(w0 + _G, 1, _G)

        tail0 = _ROUNDS * 2 * _G
        for i in range(_TAIL):
            gather_start(tail0 + i, 0, i)
        wb_wait(tail0 - _G, 1, _G)
        for i in range(_TAIL):
            gather_wait(tail0 + i, 0, i)
        wb_start(tail0, 0, _TAIL)
        wb_wait(tail0, 0, _TAIL)

    return k(W, idx1d)


def kernel(x, W):
    idx1d = x.T.astype(jnp.int32).reshape(_N)
    out = _sc_gather(W, idx1d)
    return out.reshape(_T, _B, _HIDDEN).transpose(1, 0, 2)


# speedup vs baseline: 10.5628x; 1.0002x over previous
import jax
import jax.numpy as jnp
from jax import lax
from jax.experimental import pallas as pl
from jax.experimental.pallas import tpu as pltpu
from jax.experimental.pallas import tpu_sc as plsc

_VOCAB = 100000
_HIDDEN = 128
_B, _T = 4096, 50
_N = _B * _T
_WINDOW = 128
_NWIN = _N // _WINDOW
_NW = 32
_WPW = _NWIN // _NW
_G = 3
_ROUNDS = _WPW // (2 * _G)
_TAIL = _WPW - _ROUNDS * 2 * _G


@jax.jit
def _sc_gather(W, idx1d):
    mesh = plsc.VectorSubcoreMesh(core_axis_name="core",
                                  subcore_axis_name="subcore")

    @pl.kernel(
        out_type=jax.ShapeDtypeStruct((_N, _HIDDEN), jnp.float32),
        mesh=mesh,
        scratch_types=(
            [pltpu.VMEM((_WPW * _WINDOW,), jnp.int32)]
            + [pltpu.VMEM((_G * _WINDOW, _HIDDEN), jnp.float32)
               for _ in range(2)]
            + [pltpu.SemaphoreType.DMA((2 * _G,)),
               pltpu.SemaphoreType.DMA((2,))]
        ),
    )
    def k(w_hbm, i_hbm, o_hbm, idx_v, buf_a, buf_b, gsem, osem):
        bufs = [buf_a, buf_b]
        cid = lax.axis_index("core")
        sid = lax.axis_index("subcore")
        wid = sid * 2 + cid
        base = wid * _WPW

        def gather_start(win, grp, s):
            pltpu.async_copy(
                w_hbm.at[idx_v.at[pl.ds(win * _WINDOW, _WINDOW)]],
                bufs[grp].at[pl.ds(s * _WINDOW, _WINDOW)],
                gsem.at[grp * _G + s])

        def gather_wait(win, grp, s):
            pltpu.make_async_copy(
                w_hbm.at[idx_v.at[pl.ds(win * _WINDOW, _WINDOW)]],
                bufs[grp].at[pl.ds(s * _WINDOW, _WINDOW)],
                gsem.at[grp * _G + s]).wait()

        def wb_start(win0, grp, nwin):
            pltpu.async_copy(
                bufs[grp].at[pl.ds(0, nwin * _WINDOW)],
                o_hbm.at[pl.ds((base + win0) * _WINDOW, nwin * _WINDOW)],
                osem.at[grp])

        def wb_wait(win0, grp, nwin):
            pltpu.make_async_copy(
                bufs[grp].at[pl.ds(0, nwin * _WINDOW)],
                o_hbm.at[pl.ds((base + win0) * _WINDOW, nwin * _WINDOW)],
                osem.at[grp]).wait()

        pltpu.sync_copy(i_hbm.at[pl.ds(base * _WINDOW, _WPW * _WINDOW)], idx_v)

        for s in range(_G):
            gather_start(s, 0, s)

        @pl.loop(0, _ROUNDS)
        def _(t):
            w0 = t * 2 * _G
            @pl.when(t > 0)
            def _():
                wb_wait(w0 - _G, 1, _G)
            for s in range(_G):
                gather_start(w0 + _G + s, 1, s)
            for s in range(_G):
                gather_wait(w0 + s, 0, s)
            wb_start(w0, 0, _G)
            wb_wait(w0, 0, _G)
            for s in range(_G):
                if s < _TAIL:
                    gather_start(w0 + 2 * _G + s, 0, s)
                else:
                    @pl.when(t < _ROUNDS - 1)
                    def _():
                        gather_start(w0 + 2 * _G + s, 0, s)
            for s in range(_G):
                gather_wait(w0 + _G + s, 1, s)
            wb_start(w0 + _G, 1, _G)

        tail0 = _ROUNDS * 2 * _G
        wb_wait(tail0 - _G, 1, _G)
        for i in range(_TAIL):
            gather_wait(tail0 + i, 0, i)
        wb_start(tail0, 0, _TAIL)
        wb_wait(tail0, 0, _TAIL)

    return k(W, idx1d)


def kernel(x, W):
    idx1d = x.T.astype(jnp.int32).reshape(_N)
    out = _sc_gather(W, idx1d)
    return out.reshape(_T, _B, _HIDDEN).transpose(1, 0, 2)
